# initial kernel scaffold (unmeasured)
import jax
import jax.numpy as jnp
from jax import lax
from jax.experimental import pallas as pl
from jax.experimental.pallas import tpu as pltpu

N_DEV = 4
M, N = 4096, 2048
CM = M // N_DEV
N_HOPS = N_DEV - 1


def kernel(x, w_mat):
    x = x.astype(jnp.bfloat16)
    w = w_mat.astype(jnp.bfloat16)

    def body(x_ref, w_ref, out_ref, comm_ref, sbuf_ref, send_sems, recv_sems):
        me = lax.axis_index("i")
        right = lax.rem(me + 1, N_DEV)
        left = lax.rem(me + N_DEV - 1, N_DEV)

        barrier_sem = pltpu.get_barrier_semaphore()
        for nbr in (left, right):
            pl.semaphore_signal(
                barrier_sem, inc=1,
                device_id=(nbr,), device_id_type=pl.DeviceIdType.MESH,
            )
        pl.semaphore_wait(barrier_sem, 2)

        def chunk_rows(c):
            return pl.ds(c * CM, CM)

        def partial(c):
            return jnp.dot(
                x_ref[chunk_rows(c), :], w_ref[...],
                preferred_element_type=jnp.float32,
            )

        sbuf_ref[0] = partial(me).astype(jnp.bfloat16)
        for s in range(N_HOPS):
            rdma = pltpu.make_async_remote_copy(
                src_ref=sbuf_ref.at[s % 2],
                dst_ref=comm_ref.at[s],
                send_sem=send_sems.at[s],
                recv_sem=recv_sems.at[s],
                device_id=(right,),
                device_id_type=pl.DeviceIdType.MESH,
            )
            rdma.start()
            rdma.wait()
            c = lax.rem(me + (N_HOPS - s), N_DEV)
            acc = comm_ref[s].astype(jnp.float32) + partial(c)
            if s < N_HOPS - 1:
                sbuf_ref[(s + 1) % 2] = acc.astype(jnp.bfloat16)
            else:
                out_ref[chunk_rows(lax.rem(me + 1, N_DEV)), :] = acc.astype(
                    jnp.bfloat16
                )

        for g in range(N_HOPS):
            src_c = lax.rem(me + (N_DEV + 1 - g), N_DEV)
            rdma = pltpu.make_async_remote_copy(
                src_ref=out_ref.at[chunk_rows(src_c), :],
                dst_ref=comm_ref.at[N_HOPS + g],
                send_sem=send_sems.at[N_HOPS + g],
                recv_sem=recv_sems.at[N_HOPS + g],
                device_id=(right,),
                device_id_type=pl.DeviceIdType.MESH,
            )
            rdma.start()
            rdma.wait()
            dst_c = lax.rem(me + (N_DEV - g), N_DEV)
            out_ref[chunk_rows(dst_c), :] = comm_ref[N_HOPS + g]

        amax = jnp.max(jnp.abs(out_ref[...])).astype(jnp.float32)
        scale = amax / 448.0
        inv = 1.0 / scale
        for c in range(N_DEV):
            blk = out_ref[chunk_rows(c), :].astype(jnp.float32) * inv
            q = blk.astype(jnp.float8_e4m3fn).astype(jnp.float32) * scale
            out_ref[chunk_rows(c), :] = q.astype(jnp.bfloat16)

    return pl.pallas_call(
        body,
        out_shape=jax.ShapeDtypeStruct((M, N), jnp.bfloat16),
        in_specs=[
            pl.BlockSpec(memory_space=pltpu.VMEM),
            pl.BlockSpec(memory_space=pltpu.VMEM),
        ],
        out_specs=pl.BlockSpec(memory_space=pltpu.VMEM),
        scratch_shapes=[
            pltpu.VMEM((2 * N_HOPS, CM, N), jnp.bfloat16),
            pltpu.VMEM((2, CM, N), jnp.bfloat16),
            pltpu.SemaphoreType.DMA((2 * N_HOPS,)),
            pltpu.SemaphoreType.DMA((2 * N_HOPS,)),
        ],
        compiler_params=pltpu.CompilerParams(collective_id=0),
    )(x, w)


# baseline (device time: 363148 ns/iter reference)
import jax
import jax.numpy as jnp
from jax import lax
from jax.experimental import pallas as pl
from jax.experimental.pallas import tpu as pltpu

N_DEV = 4
M, N = 4096, 2048
CM = M // N_DEV
SUB = 4
SM = CM // SUB


def kernel(x, w_mat):
    x = x.astype(jnp.bfloat16)
    w = w_mat.astype(jnp.bfloat16)

    def body(x_ref, w_ref, out_ref, comm_ref, sbuf_ref, acc_ref,
             send_sems, recv_sems, copy_sem, credit_sem):
        me = lax.axis_index("i")
        right = lax.rem(me + 1, N_DEV)
        left = lax.rem(me + N_DEV - 1, N_DEV)

        barrier_sem = pltpu.get_barrier_semaphore()
        for nbr in (left, right):
            pl.semaphore_signal(
                barrier_sem, inc=1,
                device_id=(nbr,), device_id_type=pl.DeviceIdType.MESH,
            )
        pl.semaphore_wait(barrier_sem, 2)

        def rows(c):
            return pl.ds(c * CM, CM)

        def subrows(t):
            return pl.ds(t * SM, SM)

        def compute_partial(c):
            for t in range(SUB):
                acc_ref[subrows(t), :] = jnp.dot(
                    x_ref[pl.ds(c * CM + t * SM, SM), :], w_ref[...],
                    preferred_element_type=jnp.float32,
                )

        def send(src, hop, need_credit):
            if need_credit:
                pl.semaphore_wait(credit_sem, 1)
            rdma = pltpu.make_async_remote_copy(
                src_ref=src,
                dst_ref=comm_ref.at[hop % 2],
                send_sem=send_sems.at[hop],
                recv_sem=recv_sems.at[hop],
                device_id=(right,),
                device_id_type=pl.DeviceIdType.MESH,
            )
            rdma.start()
            return rdma

        def give_credit():
            pl.semaphore_signal(
                credit_sem, inc=1,
                device_id=(left,), device_id_type=pl.DeviceIdType.MESH,
            )

        amax = jnp.float32(0.0)

        compute_partial(me)
        for t in range(SUB):
            sbuf_ref[0, subrows(t), :] = acc_ref[subrows(t), :].astype(
                jnp.bfloat16
            )
        for h in range(N_DEV - 1):
            rdma = send(sbuf_ref.at[h % 2], h, need_credit=(h >= 2))
            rdma.wait()
            c = lax.rem(me + (N_DEV - 1 - h), N_DEV)
            compute_partial(c)
            for t in range(SUB):
                acc_ref[subrows(t), :] = (
                    acc_ref[subrows(t), :]
                    + comm_ref[h % 2, subrows(t), :].astype(jnp.float32)
                )
            give_credit()
            nxt = (h + 1) % 2
            if h < N_DEV - 2:
                for t in range(SUB):
                    sbuf_ref[nxt, subrows(t), :] = acc_ref[
                        subrows(t), :
                    ].astype(jnp.bfloat16)
            else:
                for t in range(SUB):
                    sbuf_ref[1, subrows(t), :] = acc_ref[
                        subrows(t), :
                    ].astype(jnp.bfloat16)
                for t in range(SUB):
                    amax = jnp.maximum(
                        amax,
                        jnp.max(
                            jnp.abs(sbuf_ref[1, subrows(t), :]).astype(
                                jnp.float32
                            )
                        ),
                    )
                cp = pltpu.make_async_copy(
                    sbuf_ref.at[1],
                    out_ref.at[rows(lax.rem(me + 1, N_DEV)), :],
                    copy_sem,
                )
                cp.start()
                cp.wait()

        for h in range(N_DEV - 1, 2 * (N_DEV - 1)):
            g = h - (N_DEV - 1)
            src = sbuf_ref.at[1] if g == 0 else comm_ref.at[(h - 1) % 2]
            rdma = send(src, h, need_credit=True)
            rdma.wait()
            slot = h % 2
            for t in range(SUB):
                amax = jnp.maximum(
                    amax,
                    jnp.max(
                        jnp.abs(comm_ref[slot, subrows(t), :]).astype(
                            jnp.float32
                        )
                    ),
                )
            dst_c = lax.rem(me + (N_DEV - g), N_DEV)
            cp = pltpu.make_async_copy(
                comm_ref.at[slot], out_ref.at[rows(dst_c), :], copy_sem
            )
            cp.start()
            cp.wait()
            if g == 1:
                give_credit()

        scale = amax / 448.0
        inv = 1.0 / scale
        for c in range(N_DEV):
            cp = pltpu.make_async_copy(
                out_ref.at[rows(c), :], sbuf_ref.at[0], copy_sem
            )
            cp.start()
            cp.wait()
            for t in range(SUB):
                blk = sbuf_ref[0, subrows(t), :].astype(jnp.float32) * inv
                q = blk.astype(jnp.float8_e4m3fn).astype(jnp.float32) * scale
                sbuf_ref[0, subrows(t), :] = q.astype(jnp.bfloat16)
            cp = pltpu.make_async_copy(
                sbuf_ref.at[0], out_ref.at[rows(c), :], copy_sem
            )
            cp.start()
            cp.wait()

    return pl.pallas_call(
        body,
        out_shape=jax.ShapeDtypeStruct((M, N), jnp.bfloat16),
        in_specs=[
            pl.BlockSpec(memory_space=pltpu.VMEM),
            pl.BlockSpec(memory_space=pltpu.VMEM),
        ],
        out_specs=pl.BlockSpec(memory_space=pl.ANY),
        scratch_shapes=[
            pltpu.VMEM((2, CM, N), jnp.bfloat16),
            pltpu.VMEM((2, CM, N), jnp.bfloat16),
            pltpu.VMEM((CM, N), jnp.float32),
            pltpu.SemaphoreType.DMA((2 * (N_DEV - 1),)),
            pltpu.SemaphoreType.DMA((2 * (N_DEV - 1),)),
            pltpu.SemaphoreType.DMA,
            pltpu.SemaphoreType.REGULAR,
        ],
        compiler_params=pltpu.CompilerParams(collective_id=0),
    )(x, w)


# device time: 190481 ns/iter; 1.9065x vs baseline; 1.9065x over previous
import jax
import jax.numpy as jnp
from jax import lax
from jax.experimental import pallas as pl
from jax.experimental.pallas import tpu as pltpu

N_DEV = 4
M, N = 4096, 2048
CM = M // N_DEV
HALF = N // 2
SUB = 4
SM = CM // SUB
N_HOPS = N_DEV - 1


def kernel(x, w_mat):
    x = x.astype(jnp.bfloat16)
    w = w_mat.astype(jnp.bfloat16)

    def body(x_ref, w_ref, out_ref,
             commR, commL, sbufR, sbufL, accR, accL,
             my_scal, scal_ref,
             send_semsR, recv_semsR, send_semsL, recv_semsL,
             copy_semsR, copy_semsL,
             scal_send_sems, scal_recv_sems,
             creditR, creditL):
        me = lax.axis_index("i")
        right = lax.rem(me + 1, N_DEV)
        left = lax.rem(me + N_DEV - 1, N_DEV)
        diag = lax.rem(me + 2, N_DEV)

        barrier_sem = pltpu.get_barrier_semaphore()
        for nbr in (left, right):
            pl.semaphore_signal(
                barrier_sem, inc=1,
                device_id=(nbr,), device_id_type=pl.DeviceIdType.MESH,
            )
        pl.semaphore_wait(barrier_sem, 2)

        def rows(c):
            return pl.ds(c * CM, CM)

        def subrows(t):
            return pl.ds(t * SM, SM)

        R = dict(comm=commR, sbuf=sbufR, acc=accR, ssem=send_semsR,
                 rsem=recv_semsR, csem=copy_semsR, credit=creditR,
                 tgt=right, credit_to=left, col=0)
        L = dict(comm=commL, sbuf=sbufL, acc=accL, ssem=send_semsL,
                 rsem=recv_semsL, csem=copy_semsL, credit=creditL,
                 tgt=left, credit_to=right, col=HALF)

        def compute_partial(d, c):
            for t in range(SUB):
                d['acc'][subrows(t), :] = jnp.dot(
                    x_ref[pl.ds(c * CM + t * SM, SM), :],
                    w_ref[:, pl.ds(d['col'], HALF)],
                    preferred_element_type=jnp.float32,
                )

        def stage(d, slot):
            for t in range(SUB):
                d['sbuf'][slot, subrows(t), :] = d['acc'][
                    subrows(t), :
                ].astype(jnp.bfloat16)

        def send(d, src, hop):
            rdma = pltpu.make_async_remote_copy(
                src_ref=src,
                dst_ref=d['comm'].at[hop % 2],
                send_sem=d['ssem'].at[hop],
                recv_sem=d['rsem'].at[hop],
                device_id=(d['tgt'],),
                device_id_type=pl.DeviceIdType.MESH,
            )
            rdma.start()
            return rdma

        def take_credit(d):
            pl.semaphore_wait(d['credit'], 1)

        def give_credit(d):
            pl.semaphore_signal(
                d['credit'], inc=1,
                device_id=(d['credit_to'],),
                device_id_type=pl.DeviceIdType.MESH,
            )

        compute_partial(R, me)
        stage(R, 0)
        compute_partial(L, me)
        stage(L, 0)
        for h in range(N_HOPS):
            if h >= 2:
                take_credit(R)
                take_credit(L)
            rdR = send(R, sbufR.at[h % 2], h)
            rdL = send(L, sbufL.at[h % 2], h)
            cR = lax.rem(me + (N_DEV - 1 - h), N_DEV)
            cL = lax.rem(me + h + 1, N_DEV)
            compute_partial(R, cR)
            compute_partial(L, cL)
            rdR.wait()
            rdL.wait()
            for d, h_ in ((R, h), (L, h)):
                for t in range(SUB):
                    d['acc'][subrows(t), :] = (
                        d['acc'][subrows(t), :]
                        + d['comm'][h_ % 2, subrows(t), :].astype(jnp.float32)
                    )
                give_credit(d)
                stage(d, (h_ + 1) % 2)

        m = jnp.float32(0.0)
        for d in (R, L):
            for t in range(SUB):
                m = jnp.maximum(
                    m,
                    jnp.max(jnp.abs(d['sbuf'][1, subrows(t), :]).astype(
                        jnp.float32)),
                )
        my_scal[...] = jnp.full((8, 128), m, jnp.float32)
        scal_rdmas = []
        for k, tgt in enumerate((right, left, diag)):
            rd = pltpu.make_async_remote_copy(
                src_ref=my_scal,
                dst_ref=scal_ref.at[k],
                send_sem=scal_send_sems.at[k],
                recv_sem=scal_recv_sems.at[k],
                device_id=(tgt,),
                device_id_type=pl.DeviceIdType.MESH,
            )
            rd.start()
            scal_rdmas.append(rd)
        for rd in scal_rdmas:
            rd.wait()
        amax = jnp.maximum(m, jnp.max(scal_ref[...]))
        scale = amax / 448.0
        inv = 1.0 / scale

        pending = {id(R): [None, None], id(L): [None, None]}

        def quant_store(d, load, qslot, dst_c):
            if pending[id(d)][qslot] is not None:
                pending[id(d)][qslot].wait()
            for t in range(SUB):
                blk = load(t).astype(jnp.float32) * inv
                q = blk.astype(jnp.float8_e4m3fn).astype(jnp.float32) * scale
                d['sbuf'][qslot, subrows(t), :] = q.astype(jnp.bfloat16)
            cp = pltpu.make_async_copy(
                d['sbuf'].at[qslot],
                out_ref.at[rows(dst_c), pl.ds(d['col'], HALF)],
                d['csem'].at[qslot],
            )
            cp.start()
            pending[id(d)][qslot] = cp

        for g in range(N_HOPS):
            h = N_HOPS + g
            take_credit(R)
            take_credit(L)
            srcR = sbufR.at[1] if g == 0 else commR.at[(h - 1) % 2]
            srcL = sbufL.at[1] if g == 0 else commL.at[(h - 1) % 2]
            rdR = send(R, srcR, h)
            rdL = send(L, srcL, h)
            if g == 0:
                quant_store(R, lambda t: sbufR[1, subrows(t), :], 0,
                            lax.rem(me + 1, N_DEV))
                quant_store(L, lambda t: sbufL[1, subrows(t), :], 0,
                            lax.rem(me + N_DEV - 1, N_DEV))
            rdR.wait()
            rdL.wait()
            if g == 1:
                give_credit(R)
                give_credit(L)
            slot = h % 2
            dcR = lax.rem(me + (N_DEV - g), N_DEV)
            dcL = lax.rem(me + g, N_DEV)
            qs = (g + 1) % 2
            quant_store(R, lambda t: commR[slot, subrows(t), :], qs, dcR)
            quant_store(L, lambda t: commL[slot, subrows(t), :], qs, dcL)

        for d in (R, L):
            for cp in pending[id(d)]:
                if cp is not None:
                    cp.wait()

    return pl.pallas_call(
        body,
        out_shape=jax.ShapeDtypeStruct((M, N), jnp.bfloat16),
        in_specs=[
            pl.BlockSpec(memory_space=pltpu.MemorySpace.VMEM),
            pl.BlockSpec(memory_space=pltpu.MemorySpace.VMEM),
        ],
        out_specs=pl.BlockSpec(memory_space=pl.ANY),
        scratch_shapes=[
            pltpu.VMEM((2, CM, HALF), jnp.bfloat16),
            pltpu.VMEM((2, CM, HALF), jnp.bfloat16),
            pltpu.VMEM((2, CM, HALF), jnp.bfloat16),
            pltpu.VMEM((2, CM, HALF), jnp.bfloat16),
            pltpu.VMEM((CM, HALF), jnp.float32),
            pltpu.VMEM((CM, HALF), jnp.float32),
            pltpu.VMEM((8, 128), jnp.float32),
            pltpu.VMEM((3, 8, 128), jnp.float32),
            pltpu.SemaphoreType.DMA((2 * N_HOPS,)),
            pltpu.SemaphoreType.DMA((2 * N_HOPS,)),
            pltpu.SemaphoreType.DMA((2 * N_HOPS,)),
            pltpu.SemaphoreType.DMA((2 * N_HOPS,)),
            pltpu.SemaphoreType.DMA((2,)),
            pltpu.SemaphoreType.DMA((2,)),
            pltpu.SemaphoreType.DMA((3,)),
            pltpu.SemaphoreType.DMA((3,)),
            pltpu.SemaphoreType.REGULAR,
            pltpu.SemaphoreType.REGULAR,
        ],
        compiler_params=pltpu.CompilerParams(collective_id=0),
    )(x, w)


# device time: 186243 ns/iter; 1.9499x vs baseline; 1.0228x over previous
import jax
import jax.numpy as jnp
from jax import lax
from jax.experimental import pallas as pl
from jax.experimental.pallas import tpu as pltpu

N_DEV = 4
M, N = 4096, 2048
CM = M // N_DEV
HALF = N // 2
SUB = 4
SM = CM // SUB
N_HOPS = N_DEV - 1


def kernel(x, w_mat):
    x = x.astype(jnp.bfloat16)
    w = w_mat.astype(jnp.bfloat16)

    def body(x_ref, w_ref, out_ref,
             commR, commL, sbufR, sbufL, accR, accL,
             my_scal, scal_ref,
             send_semsR, recv_semsR, send_semsL, recv_semsL,
             copy_semsR, copy_semsL,
             scal_send_sems, scal_recv_sems,
             creditR, creditL):
        me = lax.axis_index("i")
        right = lax.rem(me + 1, N_DEV)
        left = lax.rem(me + N_DEV - 1, N_DEV)
        diag = lax.rem(me + 2, N_DEV)

        barrier_sem = pltpu.get_barrier_semaphore()
        for nbr in (left, right):
            pl.semaphore_signal(
                barrier_sem, inc=1,
                device_id=(nbr,), device_id_type=pl.DeviceIdType.MESH,
            )
        pl.semaphore_wait(barrier_sem, 2)

        def rows(c):
            return pl.ds(c * CM, CM)

        def subrows(t):
            return pl.ds(t * SM, SM)

        R = dict(comm=commR, sbuf=sbufR, acc=accR, ssem=send_semsR,
                 rsem=recv_semsR, csem=copy_semsR, credit=creditR,
                 tgt=right, credit_to=left, col=0)
        L = dict(comm=commL, sbuf=sbufL, acc=accL, ssem=send_semsL,
                 rsem=recv_semsL, csem=copy_semsL, credit=creditL,
                 tgt=left, credit_to=right, col=HALF)

        def compute_partial(d, c):
            for t in range(SUB):
                d['acc'][subrows(t), :] = jnp.dot(
                    x_ref[pl.ds(c * CM + t * SM, SM), :],
                    w_ref[:, pl.ds(d['col'], HALF)],
                    preferred_element_type=jnp.float32,
                )

        def send(d, src, hop):
            rdma = pltpu.make_async_remote_copy(
                src_ref=src,
                dst_ref=d['comm'].at[hop % 2],
                send_sem=d['ssem'].at[hop],
                recv_sem=d['rsem'].at[hop],
                device_id=(d['tgt'],),
                device_id_type=pl.DeviceIdType.MESH,
            )
            rdma.start()
            return rdma

        def take_credit(d):
            pl.semaphore_wait(d['credit'], 1)

        def give_credit(d):
            pl.semaphore_signal(
                d['credit'], inc=1,
                device_id=(d['credit_to'],),
                device_id_type=pl.DeviceIdType.MESH,
            )

        for t in range(SUB):
            sbufR[0, subrows(t), :] = jnp.dot(
                x_ref[pl.ds(me * CM + t * SM, SM), :],
                w_ref[:, pl.ds(0, HALF)],
                preferred_element_type=jnp.float32,
            ).astype(jnp.bfloat16)
        rdR = send(R, sbufR.at[0], 0)
        for t in range(SUB):
            sbufL[0, subrows(t), :] = jnp.dot(
                x_ref[pl.ds(me * CM + t * SM, SM), :],
                w_ref[:, pl.ds(HALF, HALF)],
                preferred_element_type=jnp.float32,
            ).astype(jnp.bfloat16)
        rdL = send(L, sbufL.at[0], 0)
        for h in range(N_HOPS):
            cR = lax.rem(me + (N_DEV - 1 - h), N_DEV)
            cL = lax.rem(me + h + 1, N_DEV)
            compute_partial(R, cR)
            compute_partial(L, cL)
            rdR.wait()
            rdL.wait()
            for d in (R, L):
                for t in range(SUB):
                    d['sbuf'][(h + 1) % 2, subrows(t), :] = (
                        d['acc'][subrows(t), :]
                        + d['comm'][h % 2, subrows(t), :].astype(jnp.float32)
                    ).astype(jnp.bfloat16)
                give_credit(d)
            if h < N_HOPS - 1:
                if h + 1 >= 2:
                    take_credit(R)
                    take_credit(L)
                rdR = send(R, sbufR.at[(h + 1) % 2], h + 1)
                rdL = send(L, sbufL.at[(h + 1) % 2], h + 1)

        pending = {id(R): [None, None], id(L): [None, None]}

        def quant_store(d, load, qslot, dst_c, scale, inv):
            if pending[id(d)][qslot] is not None:
                pending[id(d)][qslot].wait()
            for t in range(SUB):
                blk = load(t).astype(jnp.float32) * inv
                q = blk.astype(jnp.float8_e4m3fn).astype(jnp.float32) * scale
                d['sbuf'][qslot, subrows(t), :] = q.astype(jnp.bfloat16)
            cp = pltpu.make_async_copy(
                d['sbuf'].at[qslot],
                out_ref.at[rows(dst_c), pl.ds(d['col'], HALF)],
                d['csem'].at[qslot],
            )
            cp.start()
            pending[id(d)][qslot] = cp

        take_credit(R)
        take_credit(L)
        rdR = send(R, sbufR.at[1], 3)
        rdL = send(L, sbufL.at[1], 3)
        m = jnp.float32(0.0)
        for d in (R, L):
            for t in range(SUB):
                m = jnp.maximum(
                    m,
                    jnp.max(jnp.abs(d['sbuf'][1, subrows(t), :]).astype(
                        jnp.float32)),
                )
        my_scal[...] = jnp.full((8, 128), m, jnp.float32)
        scal_rdmas = []
        for k, tgt in enumerate((right, left, diag)):
            rd = pltpu.make_async_remote_copy(
                src_ref=my_scal,
                dst_ref=scal_ref.at[k],
                send_sem=scal_send_sems.at[k],
                recv_sem=scal_recv_sems.at[k],
                device_id=(tgt,),
                device_id_type=pl.DeviceIdType.MESH,
            )
            rd.start()
            scal_rdmas.append(rd)
        for rd in scal_rdmas:
            rd.wait()
        amax = jnp.maximum(m, jnp.max(scal_ref[...]))
        scale = amax / 448.0
        inv = 1.0 / scale
        quant_store(R, lambda t: sbufR[1, subrows(t), :], 0,
                    lax.rem(me + 1, N_DEV), scale, inv)
        quant_store(L, lambda t: sbufL[1, subrows(t), :], 0,
                    lax.rem(me + N_DEV - 1, N_DEV), scale, inv)
        rdR.wait()
        rdL.wait()
        take_credit(R)
        take_credit(L)
        rdR = send(R, commR.at[1], 4)
        rdL = send(L, commL.at[1], 4)
        quant_store(R, lambda t: commR[1, subrows(t), :], 1, me,
                    scale, inv)
        quant_store(L, lambda t: commL[1, subrows(t), :], 1, me,
                    scale, inv)
        rdR.wait()
        rdL.wait()
        give_credit(R)
        give_credit(L)
        take_credit(R)
        take_credit(L)
        rdR = send(R, commR.at[0], 5)
        rdL = send(L, commL.at[0], 5)
        quant_store(R, lambda t: commR[0, subrows(t), :], 0,
                    lax.rem(me + N_DEV - 1, N_DEV), scale, inv)
        quant_store(L, lambda t: commL[0, subrows(t), :], 0,
                    lax.rem(me + 1, N_DEV), scale, inv)
        rdR.wait()
        rdL.wait()
        quant_store(R, lambda t: commR[1, subrows(t), :], 1,
                    lax.rem(me + 2, N_DEV), scale, inv)
        quant_store(L, lambda t: commL[1, subrows(t), :], 1,
                    lax.rem(me + 2, N_DEV), scale, inv)

        for d in (R, L):
            for cp in pending[id(d)]:
                if cp is not None:
                    cp.wait()

    return pl.pallas_call(
        body,
        out_shape=jax.ShapeDtypeStruct((M, N), jnp.bfloat16),
        in_specs=[
            pl.BlockSpec(memory_space=pltpu.MemorySpace.VMEM),
            pl.BlockSpec(memory_space=pltpu.MemorySpace.VMEM),
        ],
        out_specs=pl.BlockSpec(memory_space=pl.ANY),
        scratch_shapes=[
            pltpu.VMEM((2, CM, HALF), jnp.bfloat16),
            pltpu.VMEM((2, CM, HALF), jnp.bfloat16),
            pltpu.VMEM((2, CM, HALF), jnp.bfloat16),
            pltpu.VMEM((2, CM, HALF), jnp.bfloat16),
            pltpu.VMEM((CM, HALF), jnp.float32),
            pltpu.VMEM((CM, HALF), jnp.float32),
            pltpu.VMEM((8, 128), jnp.float32),
            pltpu.VMEM((3, 8, 128), jnp.float32),
            pltpu.SemaphoreType.DMA((2 * N_HOPS,)),
            pltpu.SemaphoreType.DMA((2 * N_HOPS,)),
            pltpu.SemaphoreType.DMA((2 * N_HOPS,)),
            pltpu.SemaphoreType.DMA((2 * N_HOPS,)),
            pltpu.SemaphoreType.DMA((2,)),
            pltpu.SemaphoreType.DMA((2,)),
            pltpu.SemaphoreType.DMA((3,)),
            pltpu.SemaphoreType.DMA((3,)),
            pltpu.SemaphoreType.REGULAR,
            pltpu.SemaphoreType.REGULAR,
        ],
        compiler_params=pltpu.CompilerParams(collective_id=0),
    )(x, w)


# device time: 172806 ns/iter; 2.1015x vs baseline; 1.0778x over previous
import jax
import jax.numpy as jnp
from jax import lax
from jax.experimental import pallas as pl
from jax.experimental.pallas import tpu as pltpu

N_DEV = 4
M, N = 4096, 2048
CM = M // N_DEV
HALF = N // 2
SUB = 4
SM = CM // SUB
N_HOPS = 2 * (N_DEV - 1)


def kernel(x, w_mat):
    x = x.astype(jnp.bfloat16)
    w = w_mat.astype(jnp.bfloat16)

    def body(x_ref, w_ref, out_ref,
             commR, commL, sbufR, sbufL, accR, accL,
             my_scal, scal_ref,
             send_semsR, recv_semsR, send_semsL, recv_semsL,
             copy_semsR, copy_semsL,
             scal_send_sems, scal_recv_sems,
             creditR, creditL):
        me = lax.axis_index("i")
        right = lax.rem(me + 1, N_DEV)
        left = lax.rem(me + N_DEV - 1, N_DEV)
        diag = lax.rem(me + 2, N_DEV)

        barrier_sem = pltpu.get_barrier_semaphore()
        for nbr in (left, right):
            pl.semaphore_signal(
                barrier_sem, inc=1,
                device_id=(nbr,), device_id_type=pl.DeviceIdType.MESH,
            )
        pl.semaphore_wait(barrier_sem, 2)

        R = dict(comm=commR, sbuf=sbufR, acc=accR, ssem=send_semsR,
                 rsem=recv_semsR, csem=copy_semsR, credit=creditR,
                 tgt=right, credit_to=left, col=0)
        L = dict(comm=commL, sbuf=sbufL, acc=accL, ssem=send_semsL,
                 rsem=recv_semsL, csem=copy_semsL, credit=creditL,
                 tgt=left, credit_to=right, col=HALF)
        DIRS = (R, L)

        def subrows(k):
            return pl.ds(k * SM, SM)

        def xrows(c, k):
            return pl.ds(c * CM + k * SM, SM)

        def colslice(d):
            return pl.ds(d['col'], HALF)

        def chunk_R(h):
            return lax.rem(me + (N_DEV - 1 - h), N_DEV)

        def chunk_L(h):
            return lax.rem(me + h + 1, N_DEV)

        def dot_sub(d, c, k):
            d['acc'][subrows(k), :] = jnp.dot(
                x_ref[xrows(c, k), :], w_ref[:, colslice(d)],
                preferred_element_type=jnp.float32,
            )

        def rdma_send(d, hop, k, src_ref):
            rd = pltpu.make_async_remote_copy(
                src_ref=src_ref,
                dst_ref=d['comm'].at[(hop % 2) * SUB + k],
                send_sem=d['ssem'].at[hop * SUB + k],
                recv_sem=d['rsem'].at[hop * SUB + k],
                device_id=(d['tgt'],),
                device_id_type=pl.DeviceIdType.MESH,
            )
            rd.start()
            return rd

        def rdma_recv_wait(d, hop, k):
            slot = (hop % 2) * SUB + k
            pltpu.make_async_remote_copy(
                src_ref=d['comm'].at[slot],
                dst_ref=d['comm'].at[slot],
                send_sem=d['ssem'].at[hop * SUB + k],
                recv_sem=d['rsem'].at[hop * SUB + k],
                device_id=(d['tgt'],),
                device_id_type=pl.DeviceIdType.MESH,
            ).wait_recv()

        def take_credit(d):
            pl.semaphore_wait(d['credit'], 1)

        def give_credit(d):
            pl.semaphore_signal(
                d['credit'], inc=1,
                device_id=(d['credit_to'],),
                device_id_type=pl.DeviceIdType.MESH,
            )

        sends = {id(d): {h: [] for h in range(N_HOPS)} for d in DIRS}

        for k in range(SUB):
            for d in DIRS:
                d['sbuf'][k, :, :] = jnp.dot(
                    x_ref[xrows(me, k), :], w_ref[:, colslice(d)],
                    preferred_element_type=jnp.float32,
                ).astype(jnp.bfloat16)
                sends[id(d)][0].append(rdma_send(d, 0, k, d['sbuf'].at[k]))
        for k in range(SUB):
            dot_sub(R, chunk_R(0), k)
            dot_sub(L, chunk_L(0), k)

        for h in range(N_DEV - 1):
            if h >= 1:
                for d in DIRS:
                    take_credit(d)
            if h >= 1:
                for d in DIRS:
                    for rd in sends[id(d)][h - 1]:
                        rd.wait_send()
            nxt = ((h + 1) % 2) * SUB
            for k in range(SUB):
                for d in DIRS:
                    rdma_recv_wait(d, h, k)
                    slot = (h % 2) * SUB + k
                    d['sbuf'][nxt + k, :, :] = (
                        d['acc'][subrows(k), :]
                        + d['comm'][slot, :, :].astype(jnp.float32)
                    ).astype(jnp.bfloat16)
                    sends[id(d)][h + 1].append(
                        rdma_send(d, h + 1, k, d['sbuf'].at[nxt + k])
                    )
                    if h < N_DEV - 2:
                        dot_sub(d, chunk_R(h + 1) if d is R else chunk_L(h + 1), k)
            for d in DIRS:
                give_credit(d)

        m = jnp.float32(0.0)
        for d in DIRS:
            for k in range(SUB):
                m = jnp.maximum(
                    m,
                    jnp.max(jnp.abs(d['sbuf'][SUB + k, :, :]).astype(
                        jnp.float32)),
                )
        my_scal[...] = jnp.full((8, 128), m, jnp.float32)
        scal_rdmas = []
        for j, tgt in enumerate((right, left, diag)):
            rd = pltpu.make_async_remote_copy(
                src_ref=my_scal,
                dst_ref=scal_ref.at[j],
                send_sem=scal_send_sems.at[j],
                recv_sem=scal_recv_sems.at[j],
                device_id=(tgt,),
                device_id_type=pl.DeviceIdType.MESH,
            )
            rd.start()
            scal_rdmas.append(rd)
        for rd in scal_rdmas:
            rd.wait()
        amax = jnp.maximum(m, jnp.max(scal_ref[...]))
        scale = amax / 448.0
        inv = 1.0 / scale

        pend = {id(d): [None] * (2 * SUB) for d in DIRS}

        def quant_sub(d, src_slot, qflat, dst_c, k):
            if pend[id(d)][qflat] is not None:
                pend[id(d)][qflat].wait()
            blk = src_slot[:, :].astype(jnp.float32) * inv
            q = blk.astype(jnp.float8_e4m3fn).astype(jnp.float32) * scale
            d['sbuf'][qflat, :, :] = q.astype(jnp.bfloat16)
            cp = pltpu.make_async_copy(
                d['sbuf'].at[qflat],
                out_ref.at[pl.ds(dst_c * CM + k * SM, SM), colslice(d)],
                d['csem'].at[qflat],
            )
            cp.start()
            pend[id(d)][qflat] = cp

        for d in DIRS:
            for rd in sends[id(d)][2]:
                rd.wait_send()
        ownR = lax.rem(me + 1, N_DEV)
        ownL = lax.rem(me + N_DEV - 1, N_DEV)
        for k in range(SUB):
            quant_sub(R, R['sbuf'][SUB + k], k, ownR, k)
            quant_sub(L, L['sbuf'][SUB + k], k, ownL, k)

        for d in DIRS:
            take_credit(d)
            for rd in sends[id(d)][3]:
                rd.wait_send()
        for k in range(SUB):
            for d in DIRS:
                rdma_recv_wait(d, 3, k)
                slot = SUB + k
                sends[id(d)][4].append(
                    rdma_send(d, 4, k, d['comm'].at[slot])
                )
                quant_sub(d, d['comm'][slot], SUB + k, me, k)
        for d in DIRS:
            for rd in sends[id(d)][4]:
                rd.wait_send()
            give_credit(d)

        dcR4 = lax.rem(me + N_DEV - 1, N_DEV)
        dcL4 = lax.rem(me + 1, N_DEV)
        for d in DIRS:
            take_credit(d)
        for k in range(SUB):
            for d in DIRS:
                rdma_recv_wait(d, 4, k)
                slot = k
                sends[id(d)][5].append(
                    rdma_send(d, 5, k, d['comm'].at[slot])
                )
                quant_sub(d, d['comm'][slot], k,
                          dcR4 if d is R else dcL4, k)
        for d in DIRS:
            for rd in sends[id(d)][5]:
                rd.wait_send()

        dc5 = lax.rem(me + 2, N_DEV)
        for k in range(SUB):
            for d in DIRS:
                rdma_recv_wait(d, 5, k)
                quant_sub(d, d['comm'][SUB + k], SUB + k, dc5, k)

        for d in DIRS:
            for cp in pend[id(d)]:
                if cp is not None:
                    cp.wait()

    return pl.pallas_call(
        body,
        out_shape=jax.ShapeDtypeStruct((M, N), jnp.bfloat16),
        in_specs=[
            pl.BlockSpec(memory_space=pltpu.MemorySpace.VMEM),
            pl.BlockSpec(memory_space=pltpu.MemorySpace.VMEM),
        ],
        out_specs=pl.BlockSpec(memory_space=pl.ANY),
        scratch_shapes=[
            pltpu.VMEM((2 * SUB, SM, HALF), jnp.bfloat16),
            pltpu.VMEM((2 * SUB, SM, HALF), jnp.bfloat16),
            pltpu.VMEM((2 * SUB, SM, HALF), jnp.bfloat16),
            pltpu.VMEM((2 * SUB, SM, HALF), jnp.bfloat16),
            pltpu.VMEM((CM, HALF), jnp.float32),
            pltpu.VMEM((CM, HALF), jnp.float32),
            pltpu.VMEM((8, 128), jnp.float32),
            pltpu.VMEM((3, 8, 128), jnp.float32),
            pltpu.SemaphoreType.DMA((N_HOPS * SUB,)),
            pltpu.SemaphoreType.DMA((N_HOPS * SUB,)),
            pltpu.SemaphoreType.DMA((N_HOPS * SUB,)),
            pltpu.SemaphoreType.DMA((N_HOPS * SUB,)),
            pltpu.SemaphoreType.DMA((2 * SUB,)),
            pltpu.SemaphoreType.DMA((2 * SUB,)),
            pltpu.SemaphoreType.DMA((3,)),
            pltpu.SemaphoreType.DMA((3,)),
            pltpu.SemaphoreType.REGULAR,
            pltpu.SemaphoreType.REGULAR,
        ],
        compiler_params=pltpu.CompilerParams(collective_id=0),
    )(x, w)
